# baseline (speedup 1.0000x reference)
import functools

import jax
import jax.numpy as jnp
from jax import lax
from jax.experimental import pallas as pl
from jax.experimental.pallas import tpu as pltpu
from jax.experimental.pallas import tpu_sc as plsc

B = 16384
V = 100000
F = 116
D = 10
N = B * F
NG = 2 * N
TG8 = V * F * D // 8
CW = F * 16

NC = 2
NS = 16
NW = NC * NS

G = 128
K = 16
GROUPS = NG // G
GROUPS_PER_W = GROUPS // NW
STEPS = GROUPS_PER_W // K


@functools.cache
def _make_sc_gather():
    mesh = plsc.VectorSubcoreMesh(core_axis_name="c", subcore_axis_name="s")

    @functools.partial(
        pl.kernel,
        mesh=mesh,
        out_type=jax.ShapeDtypeStruct((NG, 8), jnp.float32),
        scratch_types=[
            pltpu.VMEM((K, G), jnp.int32),
            pltpu.VMEM((K * G, 8), jnp.float32),
            pltpu.SemaphoreType.DMA,
        ],
        compiler_params=pltpu.CompilerParams(use_tc_tiling_on_sc=False),
    )
    def _sc_gather(idx_hbm, table_hbm, out_hbm, idx_v, rows_v, sem):
        wid = lax.axis_index("s") * NC + lax.axis_index("c")
        base_group = wid * GROUPS_PER_W

        def step(i, carry):
            g0 = base_group + i * K
            pltpu.sync_copy(idx_hbm.at[pl.ds(g0, K)], idx_v)
            copies = [
                pltpu.async_copy(
                    table_hbm.at[idx_v.at[j]], rows_v.at[pl.ds(j * G, G)], sem
                )
                for j in range(K)
            ]
            for c in copies:
                c.wait()
            pltpu.sync_copy(rows_v, out_hbm.at[pl.ds(g0 * G, K * G)])
            return carry

        lax.fori_loop(0, STEPS, step, 0)

    return _sc_gather


BLK = 1024


def _mlp_body(cont_ref, g_ref, w1_ref, b1_ref, w2a_ref, w2e_ref, b2_ref,
              w3_ref, b3_ref, out_ref):
    cont = cont_ref[...]
    h1 = lax.dot_general(cont, w1_ref[...], (((1,), (1,)), ((), ())),
                         preferred_element_type=jnp.float32)
    h1 = jnp.maximum(h1 + b1_ref[...], 0.0)
    g = jnp.maximum(g_ref[...], 0.0)
    h2 = lax.dot_general(h1, w2a_ref[...], (((1,), (1,)), ((), ())),
                         preferred_element_type=jnp.float32)
    h2 = h2 + lax.dot_general(g, w2e_ref[...], (((1,), (1,)), ((), ())),
                              preferred_element_type=jnp.float32)
    h2 = jnp.maximum(h2 + b2_ref[...], 0.0)
    out_ref[...] = lax.dot_general(h2, w3_ref[...], (((1,), (1,)), ((), ())),
                                   preferred_element_type=jnp.float32) + b3_ref[0, 0]


def _mlp(cont_x, gathered, W1, b1, W2a, W2E, b2, W3p, b3):
    grid = (B // BLK,)
    return pl.pallas_call(
        _mlp_body,
        grid=grid,
        in_specs=[
            pl.BlockSpec((BLK, 14), lambda i: (i, 0)),
            pl.BlockSpec((BLK, CW), lambda i: (i, 0)),
            pl.BlockSpec((32, 14), lambda i: (0, 0)),
            pl.BlockSpec((1, 32), lambda i: (0, 0)),
            pl.BlockSpec((128, 32), lambda i: (0, 0)),
            pl.BlockSpec((128, CW), lambda i: (0, 0)),
            pl.BlockSpec((1, 128), lambda i: (0, 0)),
            pl.BlockSpec((128, 128), lambda i: (0, 0)),
            pl.BlockSpec((1, 1), lambda i: (0, 0)),
        ],
        out_specs=pl.BlockSpec((BLK, 128), lambda i: (i, 0)),
        out_shape=jax.ShapeDtypeStruct((B, 128), jnp.float32),
    )(cont_x, gathered, W1, b1, W2a, W2E, b2, W3p, b3)


def kernel(cat_x, cont_x, emb, W1, b1, W2, b2, W3, b3):
    cat_i = cat_x.astype(jnp.int32)
    q = (10 * jnp.arange(F, dtype=jnp.int32)) // 8
    h0 = cat_i * (F * D // 8) + q[None, :]
    idx = jnp.stack([h0, h0 + 1], axis=-1).reshape(GROUPS, G)
    table = jnp.transpose(emb, (1, 0, 2)).reshape(TG8, 8)
    rows = _make_sc_gather()(idx, table)
    gathered = rows.reshape(B, CW)
    s = (10 * jnp.arange(F)) % 8
    f_grid = jnp.arange(F)[:, None]
    d_grid = s[:, None] + jnp.arange(D)[None, :]
    W2E = (jnp.zeros((128, F, 16), jnp.float32)
           .at[:, f_grid, d_grid].set(W2[:, 32:].reshape(128, F, D))
           .reshape(128, CW))
    W3p = jnp.zeros((128, 128), jnp.float32).at[:1, :].set(W3)
    out = _mlp(cont_x, gathered,
               W1, b1.reshape(1, 32),
               W2[:, :32], W2E, b2.reshape(1, 128),
               W3p, b3.reshape(1, 1))
    return out[:, :1]


# speedup vs baseline: 1.1318x; 1.1318x over previous
import functools

import jax
import jax.numpy as jnp
from jax import lax
from jax.experimental import pallas as pl
from jax.experimental.pallas import tpu as pltpu
from jax.experimental.pallas import tpu_sc as plsc

B = 16384
V = 100000
F = 116
D = 10
N = B * F
NG = 2 * N
TG8 = V * F * D // 8
CW = F * 16

NC = 2
NS = 16
NW = NC * NS

G = 128
K = 16
GROUPS = NG // G
GROUPS_PER_W = GROUPS // NW
STEPS = GROUPS_PER_W // K


@functools.cache
def _make_sc_gather():
    mesh = plsc.VectorSubcoreMesh(core_axis_name="c", subcore_axis_name="s")

    @functools.partial(
        pl.kernel,
        mesh=mesh,
        out_type=jax.ShapeDtypeStruct((NG, 8), jnp.float32),
        scratch_types=[
            pltpu.VMEM((K, G), jnp.int32),
            pltpu.VMEM((K * G, 8), jnp.float32),
            pltpu.SemaphoreType.DMA,
        ],
        compiler_params=pltpu.CompilerParams(use_tc_tiling_on_sc=False),
    )
    def _sc_gather(idx_hbm, table_hbm, out_hbm, idx_v, rows_v, sem):
        wid = lax.axis_index("s") * NC + lax.axis_index("c")
        base_group = wid * GROUPS_PER_W

        def step(i, carry):
            g0 = base_group + i * K
            pltpu.sync_copy(idx_hbm.at[pl.ds(g0, K)], idx_v)
            copies = [
                pltpu.async_copy(
                    table_hbm.at[idx_v.at[j]], rows_v.at[pl.ds(j * G, G)], sem
                )
                for j in range(K)
            ]
            for c in copies:
                c.wait()
            pltpu.sync_copy(rows_v, out_hbm.at[pl.ds(g0 * G, K * G)])
            return carry

        lax.fori_loop(0, STEPS, step, 0)

    return _sc_gather


BLK = 1024


def _mlp_body(cont_ref, g_ref, cat_ref, w1_ref, b1_ref, w2a_ref,
              w2e0_ref, w2e1_ref, w2e2_ref, w2e3_ref, e_ref, b2_ref,
              w3_ref, b3_ref, out_ref):
    cont = cont_ref[...]
    h1 = lax.dot_general(cont, w1_ref[...], (((1,), (1,)), ((), ())),
                         preferred_element_type=jnp.float32)
    h1 = jnp.maximum(h1 + b1_ref[...], 0.0)
    h2 = lax.dot_general(h1, w2a_ref[...], (((1,), (1,)), ((), ())),
                         preferred_element_type=jnp.float32)

    vm = jnp.astype(cat_ref[...], jnp.float32)
    vm = vm - 4.0 * jnp.floor(vm * 0.25)
    m = lax.dot_general(vm, e_ref[...], (((1,), (0,)), ((), ())),
                        preferred_element_type=jnp.float32)
    g = jnp.maximum(g_ref[...], 0.0)
    for sidx, wref in enumerate((w2e0_ref, w2e1_ref, w2e2_ref, w2e3_ref)):
        gs = jnp.where(m == float(sidx), g, 0.0)
        h2 = h2 + lax.dot_general(gs, wref[...], (((1,), (1,)), ((), ())),
                                  preferred_element_type=jnp.float32)
    h2 = jnp.maximum(h2 + b2_ref[...], 0.0)
    out_ref[...] = lax.dot_general(h2, w3_ref[...], (((1,), (1,)), ((), ())),
                                   preferred_element_type=jnp.float32) + b3_ref[0, 0]


def _mlp(cont_x, gathered, cat_i, W1, b1, W2a, W2Es, E, b2, W3p, b3):
    grid = (B // BLK,)
    wspec = pl.BlockSpec((128, CW), lambda i: (0, 0))
    return pl.pallas_call(
        _mlp_body,
        grid=grid,
        in_specs=[
            pl.BlockSpec((BLK, 14), lambda i: (i, 0)),
            pl.BlockSpec((BLK, CW), lambda i: (i, 0)),
            pl.BlockSpec((BLK, F), lambda i: (i, 0)),
            pl.BlockSpec((32, 14), lambda i: (0, 0)),
            pl.BlockSpec((1, 32), lambda i: (0, 0)),
            pl.BlockSpec((128, 32), lambda i: (0, 0)),
            wspec, wspec, wspec, wspec,
            pl.BlockSpec((F, CW), lambda i: (0, 0)),
            pl.BlockSpec((1, 128), lambda i: (0, 0)),
            pl.BlockSpec((128, 128), lambda i: (0, 0)),
            pl.BlockSpec((1, 1), lambda i: (0, 0)),
        ],
        out_specs=pl.BlockSpec((BLK, 128), lambda i: (i, 0)),
        out_shape=jax.ShapeDtypeStruct((B, 128), jnp.float32),
    )(cont_x, gathered, cat_i, W1, b1, W2a, *W2Es, E, b2, W3p, b3)


def kernel(cat_x, cont_x, emb, W1, b1, W2, b2, W3, b3):
    cat_i = cat_x.astype(jnp.int32)
    h0 = ((jnp.arange(F, dtype=jnp.int32) * (V * D // 8))[None, :]
          + cat_i + cat_i // 4)
    idx = jnp.stack([h0, h0 + 1], axis=-1).reshape(GROUPS, G)
    table = emb.reshape(TG8, 8)
    rows = _make_sc_gather()(idx, table)
    gathered = rows.reshape(B, CW)

    f_grid = jnp.arange(F)[:, None]
    W2b3 = W2[:, 32:].reshape(128, F, D)
    W2Es = []
    for sidx in range(4):
        d_grid = 2 * sidx + jnp.zeros((F, 1), jnp.int32) + jnp.arange(D)[None, :]
        W2Es.append(jnp.zeros((128, F, 16), jnp.float32)
                    .at[:, f_grid, d_grid].set(W2b3).reshape(128, CW))
    E = (jnp.zeros((F, F, 16), jnp.float32)
         .at[jnp.arange(F), jnp.arange(F), :].set(1.0).reshape(F, CW))
    W3p = jnp.zeros((128, 128), jnp.float32).at[:1, :].set(W3)
    out = _mlp(cont_x, gathered, cat_i,
               W1, b1.reshape(1, 32),
               W2[:, :32], W2Es, E, b2.reshape(1, 128),
               W3p, b3.reshape(1, 1))
    return out[:, :1]
